# baseline (speedup 1.0000x reference)
import jax
import jax.numpy as jnp
from jax import lax
from jax.experimental import pallas as pl
from jax.experimental.pallas import tpu as pltpu

_N = 16384
_DIM = 768
_SD = 64
_H = 128
_K = 1024
_LR = 0.01
_BLK = 4096
_NBLK = _N // _BLK

_TRHS = (((1,), (1,)), ((), ()))


def _body(x_ref, w1_ref, b1_ref, w2_ref, b2_ref, s_ref, usage_ref,
          ns_ref, nu_ref, cnt_ref, mn_ref, acc_ref):
    i = pl.program_id(0)

    @pl.when(i == 0)
    def _init():
        acc_ref[...] = jnp.zeros_like(acc_ref)

    x = x_ref[...]
    h = jnp.maximum(
        lax.dot_general(x, w1_ref[...], _TRHS,
                        preferred_element_type=jnp.float32)
        + b1_ref[...], 0.0)
    ep = (lax.dot_general(h, w2_ref[...], _TRHS,
                          preferred_element_type=jnp.float32)
          + b2_ref[...])

    s = s_ref[...]
    n2sq = jnp.sum(s * s, axis=1, keepdims=True)
    invn2 = 1.0 / jnp.maximum(jnp.sqrt(n2sq), 1e-30)
    dot = lax.dot_general(ep, s, _TRHS,
                          preferred_element_type=jnp.float32)
    sims = dot * invn2.reshape(1, _K)

    rowmax = jnp.max(sims, axis=1, keepdims=True)
    onehot = (sims == rowmax).astype(jnp.bfloat16)

    rhs = jnp.concatenate(
        [ep.astype(jnp.bfloat16), jnp.ones((_BLK, _SD), jnp.bfloat16)],
        axis=1)
    acc_ref[...] += lax.dot_general(
        onehot, rhs, (((0,), (0,)), ((), ())),
        preferred_element_type=jnp.float32)

    @pl.when(i == _NBLK - 1)
    def _finish():
        acc = acc_ref[...]
        lane_k = lax.broadcasted_iota(jnp.int32, (_K, _H), 1)
        sums = acc[:, :_SD]
        counts = (jnp.sum(jnp.where(lane_k >= _SD, acc, 0.0), axis=1,
                          keepdims=True) * (1.0 / _SD))
        maxc = jnp.maximum(counts, 1.0)
        active = counts > 0.0
        delta = jnp.where(active, _LR * (sums / maxc - s_ref[...]), 0.0)
        ns_ref[...] = s_ref[...] + delta
        nu_ref[...] = usage_ref[...] + counts
        nrm = jnp.sqrt(jnp.sum(delta * delta, axis=1, keepdims=True))
        num_up = jnp.sum(active.astype(jnp.float32), axis=0,
                         keepdims=True)
        cnt_ref[...] = num_up.astype(jnp.int32)
        mn_ref[...] = (jnp.sum(jnp.where(active, nrm, 0.0), axis=0,
                               keepdims=True)
                       / jnp.maximum(num_up, 1.0))


def kernel(episodic_traces, W1, b1, W2, b2, schemas, schema_usage):
    f32 = jnp.float32
    const = lambda *_: (0, 0)
    out = pl.pallas_call(
        _body,
        grid=(_NBLK,),
        in_specs=[
            pl.BlockSpec((_BLK, _DIM), lambda i: (i, 0)),
            pl.BlockSpec((_H, _DIM), const),
            pl.BlockSpec((_H,), lambda *_: (0,)),
            pl.BlockSpec((_SD, _H), const),
            pl.BlockSpec((_SD,), lambda *_: (0,)),
            pl.BlockSpec((_K, _SD), const),
            pl.BlockSpec((_K, 1), const),
        ],
        out_specs=[
            pl.BlockSpec((_K, _SD), const),
            pl.BlockSpec((_K, 1), const),
            pl.BlockSpec((1, 1), const),
            pl.BlockSpec((1, 1), const),
        ],
        out_shape=[
            jax.ShapeDtypeStruct((_K, _SD), f32),
            jax.ShapeDtypeStruct((_K, 1), f32),
            jax.ShapeDtypeStruct((1, 1), jnp.int32),
            jax.ShapeDtypeStruct((1, 1), f32),
        ],
        scratch_shapes=[pltpu.VMEM((_K, _H), f32)],
    )(episodic_traces, W1, b1, W2, b2, schemas, schema_usage[:, None])
    ns, nu2, cnt, mn = out
    return (ns, nu2[:, 0], cnt[0, 0], mn[0, 0])


# speedup vs baseline: 1.0067x; 1.0067x over previous
import jax
import jax.numpy as jnp
from jax import lax
from jax.experimental import pallas as pl
from jax.experimental.pallas import tpu as pltpu

_N = 16384
_DIM = 768
_SD = 64
_H = 128
_K = 1024
_LR = 0.01
_BLK = 2048
_NBLK = _N // _BLK

_TRHS = (((1,), (1,)), ((), ()))


def _body(x_ref, w1_ref, b1_ref, w2_ref, b2_ref, s_ref, usage_ref,
          ns_ref, nu_ref, cnt_ref, mn_ref, acc_ref):
    i = pl.program_id(0)

    @pl.when(i == 0)
    def _init():
        acc_ref[...] = jnp.zeros_like(acc_ref)

    x = x_ref[...]
    h = jnp.maximum(
        lax.dot_general(x, w1_ref[...], _TRHS,
                        preferred_element_type=jnp.float32)
        + b1_ref[...], 0.0)
    ep = (lax.dot_general(h, w2_ref[...], _TRHS,
                          preferred_element_type=jnp.float32)
          + b2_ref[...])

    s = s_ref[...]
    n2sq = jnp.sum(s * s, axis=1, keepdims=True)
    invn2 = 1.0 / jnp.maximum(jnp.sqrt(n2sq), 1e-30)
    dot = lax.dot_general(ep, s, _TRHS,
                          preferred_element_type=jnp.float32)
    sims = dot * invn2.reshape(1, _K)

    rowmax = jnp.max(sims, axis=1, keepdims=True)
    onehot = (sims == rowmax).astype(jnp.bfloat16)

    rhs = jnp.concatenate(
        [ep.astype(jnp.bfloat16), jnp.ones((_BLK, _SD), jnp.bfloat16)],
        axis=1)
    acc_ref[...] += lax.dot_general(
        onehot, rhs, (((0,), (0,)), ((), ())),
        preferred_element_type=jnp.float32)

    @pl.when(i == _NBLK - 1)
    def _finish():
        acc = acc_ref[...]
        lane_k = lax.broadcasted_iota(jnp.int32, (_K, _H), 1)
        sums = acc[:, :_SD]
        counts = (jnp.sum(jnp.where(lane_k >= _SD, acc, 0.0), axis=1,
                          keepdims=True) * (1.0 / _SD))
        maxc = jnp.maximum(counts, 1.0)
        active = counts > 0.0
        delta = jnp.where(active, _LR * (sums / maxc - s_ref[...]), 0.0)
        ns_ref[...] = s_ref[...] + delta
        nu_ref[...] = usage_ref[...] + counts
        nrm = jnp.sqrt(jnp.sum(delta * delta, axis=1, keepdims=True))
        num_up = jnp.sum(active.astype(jnp.float32), axis=0,
                         keepdims=True)
        cnt_ref[...] = num_up.astype(jnp.int32)
        mn_ref[...] = (jnp.sum(jnp.where(active, nrm, 0.0), axis=0,
                               keepdims=True)
                       / jnp.maximum(num_up, 1.0))


def kernel(episodic_traces, W1, b1, W2, b2, schemas, schema_usage):
    f32 = jnp.float32
    const = lambda *_: (0, 0)
    out = pl.pallas_call(
        _body,
        grid=(_NBLK,),
        in_specs=[
            pl.BlockSpec((_BLK, _DIM), lambda i: (i, 0)),
            pl.BlockSpec((_H, _DIM), const),
            pl.BlockSpec((_H,), lambda *_: (0,)),
            pl.BlockSpec((_SD, _H), const),
            pl.BlockSpec((_SD,), lambda *_: (0,)),
            pl.BlockSpec((_K, _SD), const),
            pl.BlockSpec((_K, 1), const),
        ],
        out_specs=[
            pl.BlockSpec((_K, _SD), const),
            pl.BlockSpec((_K, 1), const),
            pl.BlockSpec((1, 1), const),
            pl.BlockSpec((1, 1), const),
        ],
        out_shape=[
            jax.ShapeDtypeStruct((_K, _SD), f32),
            jax.ShapeDtypeStruct((_K, 1), f32),
            jax.ShapeDtypeStruct((1, 1), jnp.int32),
            jax.ShapeDtypeStruct((1, 1), f32),
        ],
        scratch_shapes=[pltpu.VMEM((_K, _H), f32)],
    )(episodic_traces, W1, b1, W2, b2, schemas, schema_usage[:, None])
    ns, nu2, cnt, mn = out
    return (ns, nu2[:, 0], cnt[0, 0], mn[0, 0])
